# baseline (speedup 1.0000x reference)
import functools

import jax
import jax.numpy as jnp
from jax import lax
from jax.experimental import pallas as pl
from jax.experimental.pallas import tpu as pltpu
from jax.experimental.pallas import tpu_sc as plsc

NC = 2
NS = 16
NW = NC * NS


def _node_body(x_ref, w_ref, asrc_ref, adst_ref, xw_ref, as_ref, ad_ref):
    xw = jnp.dot(x_ref[...], w_ref[...], preferred_element_type=jnp.float32)
    xw_ref[...] = xw
    as_ref[...] = jnp.dot(xw, asrc_ref[...], preferred_element_type=jnp.float32)
    ad_ref[...] = jnp.dot(xw, adst_ref[...], preferred_element_type=jnp.float32)


def _node_scores(x, W, att_src, att_dst, bn):
    N, D = x.shape
    grid = (N // bn,)
    return pl.pallas_call(
        _node_body,
        grid=grid,
        in_specs=[
            pl.BlockSpec((bn, D), lambda g: (g, 0)),
            pl.BlockSpec((D, D), lambda g: (0, 0)),
            pl.BlockSpec((D, 1), lambda g: (0, 0)),
            pl.BlockSpec((D, 1), lambda g: (0, 0)),
        ],
        out_specs=[
            pl.BlockSpec((bn, D), lambda g: (g, 0)),
            pl.BlockSpec((bn, 1), lambda g: (g, 0)),
            pl.BlockSpec((bn, 1), lambda g: (g, 0)),
        ],
        out_shape=[
            jax.ShapeDtypeStruct((N, D), jnp.float32),
            jax.ShapeDtypeStruct((N, 1), jnp.float32),
            jax.ShapeDtypeStruct((N, 1), jnp.float32),
        ],
    )(x, W, att_src.reshape(D, 1), att_dst.reshape(D, 1))


def _edge_body(CB, eat_ref, we_ref, ae_ref, out_ref):
    g = pl.program_id(0)
    v2 = jnp.dot(we_ref[...], ae_ref[...], preferred_element_type=jnp.float32)
    s = lax.dot_general(v2, eat_ref[...], (((0,), (0,)), ((), ())),
                        preferred_element_type=jnp.float32)
    rpc = CB // 128
    for m in range(rpc):
        out_ref[pl.ds(g * rpc + m, 1), :] = s[:, m * 128:(m + 1) * 128]


def _edge_scores(edge_attr_t, W_edge, att_edge, cb):
    DE, E = edge_attr_t.shape
    D = W_edge.shape[1]
    return pl.pallas_call(
        functools.partial(_edge_body, cb),
        grid=(E // cb,),
        in_specs=[
            pl.BlockSpec((DE, cb), lambda g: (0, g)),
            pl.BlockSpec((DE, D), lambda g: (0, 0)),
            pl.BlockSpec((D, 1), lambda g: (0, 0)),
        ],
        out_specs=pl.BlockSpec((E // 128, 128), lambda g: (0, 0)),
        out_shape=jax.ShapeDtypeStruct((E // 128, 128), jnp.float32),
    )(edge_attr_t, W_edge, att_edge.reshape(D, 1))


def _sc_edge_kernel(N, D, E, K):
    EPW = E // NW
    P1 = EPW // 16
    NCHUNK = EPW // K
    RPT = (N // NS) // 8 * 8
    REM = N - NS * RPT
    ZB = K

    mesh = plsc.VectorSubcoreMesh(core_axis_name="c", subcore_axis_name="s")

    @functools.partial(
        pl.kernel,
        out_type=(
            jax.ShapeDtypeStruct((NC, N, D), jnp.float32),
            jax.ShapeDtypeStruct((NC, N), jnp.float32),
            jax.ShapeDtypeStruct((NC, N), jnp.float32),
            jax.ShapeDtypeStruct((NC, N), jnp.float32),
        ),
        mesh=mesh,
        scratch_types=[
            pltpu.VMEM((K,), jnp.int32),
            pltpu.VMEM((K,), jnp.int32),
            pltpu.VMEM((K,), jnp.int32),
            pltpu.VMEM((K,), jnp.int32),
            pltpu.VMEM((K,), jnp.float32),
            pltpu.VMEM((K,), jnp.float32),
            pltpu.VMEM((K,), jnp.int32),
            pltpu.VMEM((K,), jnp.int32),
            pltpu.VMEM((K,), jnp.float32),
            pltpu.VMEM((K,), jnp.float32),
            pltpu.VMEM((N,), jnp.float32),
            pltpu.VMEM((N,), jnp.float32),
            pltpu.VMEM((K,), jnp.float32),
            pltpu.VMEM((K,), jnp.float32),
            pltpu.VMEM((K,), jnp.float32),
            pltpu.VMEM((RPT,), jnp.float32),
            pltpu.VMEM((2, K, D), jnp.float32),
            pltpu.VMEM_SHARED((N, D), jnp.float32),
            pltpu.VMEM_SHARED((N,), jnp.float32),
            pltpu.VMEM_SHARED((N,), jnp.float32),
            pltpu.VMEM_SHARED((N,), jnp.float32),
            pltpu.SemaphoreType.DMA,
            pltpu.SemaphoreType.DMA,
            pltpu.SemaphoreType.DMA,
            pltpu.SemaphoreType.DMA,
            pltpu.SemaphoreType.DMA,
            pltpu.SemaphoreType.DMA,
        ],
        compiler_params=pltpu.CompilerParams(needs_layout_passes=False),
    )
    def body(src_h, dst_h, asrc_h, adst_h, ae0_h, xw_h,
             num_h, den_h, cnt_h, sa0_h,
             srcc0, srcc1, dstc0, dstc1, ae0c0, ae0c1, dsts0, dsts1,
             ae0s0, ae0s1, asrc_v, adst_v, exc0, exc1, ones_v, zer_v,
             rows, num_s, den_s, cnt_s, sa0_s,
             sg0, sg1, gg0, gg1, ss0, ss1):
        cid = lax.axis_index("c")
        sid = lax.axis_index("s")
        wid = cid * NS + sid
        base = wid * EPW
        srcc = (srcc0, srcc1)
        dstc = (dstc0, dstc1)
        ae0c = (ae0c0, ae0c1)
        dsts = (dsts0, dsts1)
        ae0s = (ae0s0, ae0s1)
        exc = (exc0, exc1)
        sgs = (sg0, sg1)
        ggs = (gg0, gg1)
        sss = (ss0, ss1)

        zero16 = jnp.zeros((16,), jnp.float32)
        one16 = jnp.ones((16,), jnp.float32)

        def _zinit(j, _):
            zer_v[pl.ds(j * 16, 16)] = zero16
            return 0
        lax.fori_loop(0, RPT // 16, _zinit, 0)
        for q in range(K // 16):
            ones_v[pl.ds(q * 16, 16)] = one16

        def _zrow(e, _):
            for h in range(D // 16):
                rows[0, e, pl.ds(h * 16, 16)] = zero16
            return 0
        lax.fori_loop(0, K, _zrow, 0)

        pltpu.sync_copy(asrc_h, asrc_v)
        pltpu.sync_copy(adst_h, adst_v)

        r0 = sid * RPT
        for q in range((RPT + ZB - 1) // ZB):
            sz = min(ZB, RPT - q * ZB)
            pltpu.sync_copy(rows.at[0, pl.ds(0, sz)],
                            num_s.at[pl.ds(r0 + q * ZB, sz)])
        pltpu.sync_copy(zer_v, den_s.at[pl.ds(r0, RPT)])
        pltpu.sync_copy(zer_v, cnt_s.at[pl.ds(r0, RPT)])
        pltpu.sync_copy(zer_v, sa0_s.at[pl.ds(r0, RPT)])

        @pl.when(sid == NS - 1)
        def _():
            pltpu.sync_copy(rows.at[0, pl.ds(0, REM)],
                            num_s.at[pl.ds(NS * RPT, REM)])
            pltpu.sync_copy(zer_v.at[pl.ds(0, REM)],
                            den_s.at[pl.ds(NS * RPT, REM)])
            pltpu.sync_copy(zer_v.at[pl.ds(0, REM)],
                            cnt_s.at[pl.ds(NS * RPT, REM)])
            pltpu.sync_copy(zer_v.at[pl.ds(0, REM)],
                            sa0_s.at[pl.ds(NS * RPT, REM)])
        plsc.subcore_barrier()

        def _stage(c, b):
            gb = base + c * K
            pltpu.async_copy(src_h.at[pl.ds(gb, K)], srcc[b], sgs[b])
            pltpu.async_copy(dst_h.at[pl.ds(gb, K)], dstc[b], sgs[b])
            pltpu.async_copy(ae0_h.at[pl.ds(gb, K)], ae0c[b], sgs[b])

        def _wait_stage(b):
            pltpu.make_async_copy(src_h.at[pl.ds(0, K)], srcc[b], sgs[b]).wait()
            pltpu.make_async_copy(dst_h.at[pl.ds(0, K)], dstc[b], sgs[b]).wait()
            pltpu.make_async_copy(ae0_h.at[pl.ds(0, K)], ae0c[b], sgs[b]).wait()

        def _gather(b):
            pltpu.async_copy(xw_h.at[srcc[b]], rows.at[b], ggs[b])

        def _wait_gather(b):
            pltpu.make_async_copy(xw_h.at[srcc[b]], rows.at[b], ggs[b]).wait()

        def _wait_scatter(b):
            pltpu.make_async_copy(rows.at[b], num_s.at[dsts[b]], sss[b]).wait()
            pltpu.make_async_copy(exc[b], den_s.at[dsts[b]], sss[b]).wait()
            pltpu.make_async_copy(ones_v, cnt_s.at[dsts[b]], sss[b]).wait()
            pltpu.make_async_copy(ae0s[b], sa0_s.at[dsts[b]], sss[b]).wait()

        def _compute(b):
            for q in range(K // 16):
                sl = pl.ds(q * 16, 16)
                al = (plsc.load_gather(asrc_v, [srcc[b][sl]])
                      + plsc.load_gather(adst_v, [dstc[b][sl]]) + ae0c[b][sl])
                al = jnp.where(al > 0, al, 0.2 * al)
                e16 = jnp.exp(al)
                exc[b][sl] = e16
                dsts[b][sl] = dstc[b][sl]
                ae0s[b][sl] = ae0c[b][sl]
                for t in range(16):
                    e = q * 16 + t
                    w = e16[t]
                    for h in range(D // 16):
                        sl2 = pl.ds(h * 16, 16)
                        rows[b, e, sl2] = rows[b, e, sl2] * w

        def _scatter(b):
            pltpu.async_copy(rows.at[b], num_s.at[dsts[b]], sss[b], add=True)
            pltpu.async_copy(exc[b], den_s.at[dsts[b]], sss[b], add=True)
            pltpu.async_copy(ones_v, cnt_s.at[dsts[b]], sss[b], add=True)
            pltpu.async_copy(ae0s[b], sa0_s.at[dsts[b]], sss[b], add=True)

        _stage(0, 0)
        _stage(1, 1)
        _wait_stage(0)
        _gather(0)

        def _step(c, b, in_loop):
            nb = 1 - b
            if in_loop:
                _wait_stage(nb)

                @pl.when(c >= 1)
                def _():
                    _wait_scatter(nb)
                _gather(nb)
            _wait_gather(b)
            _compute(b)
            if in_loop:
                @pl.when(c <= NCHUNK - 3)
                def _():
                    _stage(c + 2, b)
            _scatter(b)

        def _pair(c2, _):
            for b in range(2):
                c = c2 * 2 + b
                _step(c, b, True)
            return 0
        lax.fori_loop(0, NCHUNK // 2, _pair, 0)
        _step(NCHUNK - 1, (NCHUNK - 1) % 2, False)
        _wait_scatter(0)
        _wait_scatter(1)

        plsc.subcore_barrier()

        pltpu.sync_copy(num_s.at[pl.ds(r0, RPT)],
                        num_h.at[cid, pl.ds(r0, RPT)])

        @pl.when(sid == NS - 1)
        def _():
            pltpu.sync_copy(num_s.at[pl.ds(NS * RPT, REM)],
                            num_h.at[cid, pl.ds(NS * RPT, REM)])

        @pl.when(sid == 0)
        def _():
            pltpu.sync_copy(den_s, den_h.at[cid])
            pltpu.sync_copy(cnt_s, cnt_h.at[cid])
            pltpu.sync_copy(sa0_s, sa0_h.at[cid])

    return body


def _merge_body(num_ref, den_ref, cnt_ref, sa0_ref, xw_ref, as_ref, ad_ref,
                b_ref, g_ref, be_ref, out_ref):
    den = den_ref[...]
    cnt = cnt_ref[...]
    sa0 = sa0_ref[...]
    ael = sa0 / jnp.maximum(cnt, 1.0)
    al = as_ref[...] + ad_ref[...] + ael
    al = jnp.where(al > 0, al, 0.2 * al)
    exl = jnp.exp(al)
    num = num_ref[0] + num_ref[1] + exl * xw_ref[...]
    out = num / (den + exl + 1e-16) + b_ref[...]
    mu = jnp.mean(out, axis=1, keepdims=True)
    var = jnp.mean((out - mu) ** 2, axis=1, keepdims=True)
    out_ref[...] = (out - mu) / jnp.sqrt(var + 1e-5) * g_ref[...] + be_ref[...]


def _merge(num, den, cnt, sa0, xw, a_src, a_dst, bias, gamma, beta, bn):
    _, N, D = num.shape
    den3 = den.sum(axis=0).reshape(N, 1)
    cnt3 = cnt.sum(axis=0).reshape(N, 1)
    sa03 = sa0.sum(axis=0).reshape(N, 1)
    return pl.pallas_call(
        _merge_body,
        grid=(N // bn,),
        in_specs=[
            pl.BlockSpec((NC, bn, D), lambda g: (0, g, 0)),
            pl.BlockSpec((bn, 1), lambda g: (g, 0)),
            pl.BlockSpec((bn, 1), lambda g: (g, 0)),
            pl.BlockSpec((bn, 1), lambda g: (g, 0)),
            pl.BlockSpec((bn, D), lambda g: (g, 0)),
            pl.BlockSpec((bn, 1), lambda g: (g, 0)),
            pl.BlockSpec((bn, 1), lambda g: (g, 0)),
            pl.BlockSpec((1, D), lambda g: (0, 0)),
            pl.BlockSpec((1, D), lambda g: (0, 0)),
            pl.BlockSpec((1, D), lambda g: (0, 0)),
        ],
        out_specs=pl.BlockSpec((bn, D), lambda g: (g, 0)),
        out_shape=jax.ShapeDtypeStruct((N, D), jnp.float32),
    )(num, den3, cnt3, sa03, xw, a_src, a_dst,
      bias.reshape(1, D), gamma.reshape(1, D), beta.reshape(1, D))


def kernel(x, edge_index, edge_attr, W, att_src, att_dst, W_edge, att_edge,
           bias, gamma, beta):
    N, D = x.shape
    E = edge_index.shape[1]

    src = edge_index[0].astype(jnp.int32)
    dst = edge_index[1].astype(jnp.int32)

    xw, a_src, a_dst = _node_scores(x, W, att_src, att_dst, bn=2000)
    ae0 = _edge_scores(edge_attr.T, W_edge, att_edge, cb=2560)

    sc = _sc_edge_kernel(N, D, E, K=80)
    num, den, cnt, sa0 = sc(src, dst, a_src.reshape(N), a_dst.reshape(N),
                            ae0.reshape(E), xw)

    return _merge(num, den, cnt, sa0, xw, a_src, a_dst,
                  bias, gamma, beta, bn=2000)


# speedup vs baseline: 1.1471x; 1.1471x over previous
import functools

import jax
import jax.numpy as jnp
from jax import lax
from jax.experimental import pallas as pl
from jax.experimental.pallas import tpu as pltpu
from jax.experimental.pallas import tpu_sc as plsc

NC = 2
NS = 16
NW = NC * NS


def _node_body(x_ref, w_ref, asrc_ref, adst_ref, xw_ref, as_ref, ad_ref):
    xw = jnp.dot(x_ref[...], w_ref[...], preferred_element_type=jnp.float32)
    xw_ref[...] = xw
    as_ref[...] = jnp.dot(xw, asrc_ref[...], preferred_element_type=jnp.float32)
    ad_ref[...] = jnp.dot(xw, adst_ref[...], preferred_element_type=jnp.float32)


def _node_scores(x, W, att_src, att_dst, bn):
    N, D = x.shape
    grid = (N // bn,)
    return pl.pallas_call(
        _node_body,
        grid=grid,
        in_specs=[
            pl.BlockSpec((bn, D), lambda g: (g, 0)),
            pl.BlockSpec((D, D), lambda g: (0, 0)),
            pl.BlockSpec((D, 1), lambda g: (0, 0)),
            pl.BlockSpec((D, 1), lambda g: (0, 0)),
        ],
        out_specs=[
            pl.BlockSpec((bn, D), lambda g: (g, 0)),
            pl.BlockSpec((bn, 1), lambda g: (g, 0)),
            pl.BlockSpec((bn, 1), lambda g: (g, 0)),
        ],
        out_shape=[
            jax.ShapeDtypeStruct((N, D), jnp.float32),
            jax.ShapeDtypeStruct((N, 1), jnp.float32),
            jax.ShapeDtypeStruct((N, 1), jnp.float32),
        ],
    )(x, W, att_src.reshape(D, 1), att_dst.reshape(D, 1))


def _edge_body(E, CB, eat_ref, we_ref, ae_ref, out_ref):
    v2 = jnp.dot(we_ref[...], ae_ref[...], preferred_element_type=jnp.float32)
    rpc = CB // 128

    def step(c, _):
        chunk = eat_ref[:, pl.ds(c * CB, CB)]
        s = lax.dot_general(v2, chunk, (((0,), (0,)), ((), ())),
                            preferred_element_type=jnp.float32)
        for m in range(rpc):
            out_ref[pl.ds(c * rpc + m, 1), :] = s[:, m * 128:(m + 1) * 128]
        return 0
    lax.fori_loop(0, E // CB, step, 0)


def _edge_scores(edge_attr_t, W_edge, att_edge, cb):
    DE, E = edge_attr_t.shape
    D = W_edge.shape[1]
    return pl.pallas_call(
        functools.partial(_edge_body, E, cb),
        in_specs=[
            pl.BlockSpec((DE, E), lambda: (0, 0)),
            pl.BlockSpec((DE, D), lambda: (0, 0)),
            pl.BlockSpec((D, 1), lambda: (0, 0)),
        ],
        out_specs=pl.BlockSpec((E // 128, 128), lambda: (0, 0)),
        out_shape=jax.ShapeDtypeStruct((E // 128, 128), jnp.float32),
    )(edge_attr_t, W_edge, att_edge.reshape(D, 1))


def _sc_edge_kernel(N, D, E, K):
    EPW = E // NW
    P1 = EPW // 16
    NCHUNK = EPW // K
    RPT = (N // NS) // 8 * 8
    REM = N - NS * RPT
    ZB = K

    mesh = plsc.VectorSubcoreMesh(core_axis_name="c", subcore_axis_name="s")

    @functools.partial(
        pl.kernel,
        out_type=(
            jax.ShapeDtypeStruct((NC, N, D), jnp.float32),
            jax.ShapeDtypeStruct((NC, N), jnp.float32),
            jax.ShapeDtypeStruct((NC, N), jnp.float32),
            jax.ShapeDtypeStruct((NC, N), jnp.float32),
        ),
        mesh=mesh,
        scratch_types=[
            pltpu.VMEM((K,), jnp.int32),
            pltpu.VMEM((K,), jnp.int32),
            pltpu.VMEM((K,), jnp.int32),
            pltpu.VMEM((K,), jnp.int32),
            pltpu.VMEM((K,), jnp.float32),
            pltpu.VMEM((K,), jnp.float32),
            pltpu.VMEM((K,), jnp.int32),
            pltpu.VMEM((K,), jnp.int32),
            pltpu.VMEM((K,), jnp.float32),
            pltpu.VMEM((K,), jnp.float32),
            pltpu.VMEM((N,), jnp.float32),
            pltpu.VMEM((N,), jnp.float32),
            pltpu.VMEM((K,), jnp.float32),
            pltpu.VMEM((K,), jnp.float32),
            pltpu.VMEM((K,), jnp.float32),
            pltpu.VMEM((RPT,), jnp.float32),
            pltpu.VMEM((2, K, D), jnp.float32),
            pltpu.VMEM_SHARED((N, D), jnp.float32),
            pltpu.VMEM_SHARED((N,), jnp.float32),
            pltpu.VMEM_SHARED((N,), jnp.float32),
            pltpu.VMEM_SHARED((N,), jnp.float32),
            pltpu.SemaphoreType.DMA,
            pltpu.SemaphoreType.DMA,
            pltpu.SemaphoreType.DMA,
            pltpu.SemaphoreType.DMA,
            pltpu.SemaphoreType.DMA,
            pltpu.SemaphoreType.DMA,
        ],
        compiler_params=pltpu.CompilerParams(needs_layout_passes=False),
    )
    def body(src_h, dst_h, asrc_h, adst_h, ae0_h, xw_h,
             num_h, den_h, cnt_h, sa0_h,
             srcc0, srcc1, dstc0, dstc1, ae0c0, ae0c1, dsts0, dsts1,
             ae0s0, ae0s1, asrc_v, adst_v, exc0, exc1, ones_v, zer_v,
             rows, num_s, den_s, cnt_s, sa0_s,
             sg0, sg1, gg0, gg1, ss0, ss1):
        cid = lax.axis_index("c")
        sid = lax.axis_index("s")
        wid = cid * NS + sid
        base = wid * EPW
        srcc = (srcc0, srcc1)
        dstc = (dstc0, dstc1)
        ae0c = (ae0c0, ae0c1)
        dsts = (dsts0, dsts1)
        ae0s = (ae0s0, ae0s1)
        exc = (exc0, exc1)
        sgs = (sg0, sg1)
        ggs = (gg0, gg1)
        sss = (ss0, ss1)

        zero16 = jnp.zeros((16,), jnp.float32)
        one16 = jnp.ones((16,), jnp.float32)

        def _zinit(j, _):
            zer_v[pl.ds(j * 16, 16)] = zero16
            return 0
        lax.fori_loop(0, RPT // 16, _zinit, 0)
        for q in range(K // 16):
            ones_v[pl.ds(q * 16, 16)] = one16

        def _zrow(e, _):
            for h in range(D // 16):
                rows[0, e, pl.ds(h * 16, 16)] = zero16
            return 0
        lax.fori_loop(0, K, _zrow, 0)

        pltpu.sync_copy(asrc_h, asrc_v)
        pltpu.sync_copy(adst_h, adst_v)

        r0 = sid * RPT
        for q in range((RPT + ZB - 1) // ZB):
            sz = min(ZB, RPT - q * ZB)
            pltpu.sync_copy(rows.at[0, pl.ds(0, sz)],
                            num_s.at[pl.ds(r0 + q * ZB, sz)])
        pltpu.sync_copy(zer_v, den_s.at[pl.ds(r0, RPT)])
        pltpu.sync_copy(zer_v, cnt_s.at[pl.ds(r0, RPT)])
        pltpu.sync_copy(zer_v, sa0_s.at[pl.ds(r0, RPT)])

        @pl.when(sid == NS - 1)
        def _():
            pltpu.sync_copy(rows.at[0, pl.ds(0, REM)],
                            num_s.at[pl.ds(NS * RPT, REM)])
            pltpu.sync_copy(zer_v.at[pl.ds(0, REM)],
                            den_s.at[pl.ds(NS * RPT, REM)])
            pltpu.sync_copy(zer_v.at[pl.ds(0, REM)],
                            cnt_s.at[pl.ds(NS * RPT, REM)])
            pltpu.sync_copy(zer_v.at[pl.ds(0, REM)],
                            sa0_s.at[pl.ds(NS * RPT, REM)])
        plsc.subcore_barrier()

        def _stage(c, b):
            gb = base + c * K
            pltpu.async_copy(src_h.at[pl.ds(gb, K)], srcc[b], sgs[b])
            pltpu.async_copy(dst_h.at[pl.ds(gb, K)], dstc[b], sgs[b])
            pltpu.async_copy(ae0_h.at[pl.ds(gb, K)], ae0c[b], sgs[b])

        def _wait_stage(b):
            pltpu.make_async_copy(src_h.at[pl.ds(0, K)], srcc[b], sgs[b]).wait()
            pltpu.make_async_copy(dst_h.at[pl.ds(0, K)], dstc[b], sgs[b]).wait()
            pltpu.make_async_copy(ae0_h.at[pl.ds(0, K)], ae0c[b], sgs[b]).wait()

        def _gather(b):
            pltpu.async_copy(xw_h.at[srcc[b]], rows.at[b], ggs[b])

        def _wait_gather(b):
            pltpu.make_async_copy(xw_h.at[srcc[b]], rows.at[b], ggs[b]).wait()

        def _wait_scatter(b):
            pltpu.make_async_copy(rows.at[b], num_s.at[dsts[b]], sss[b]).wait()
            pltpu.make_async_copy(exc[b], den_s.at[dsts[b]], sss[b]).wait()
            pltpu.make_async_copy(ones_v, cnt_s.at[dsts[b]], sss[b]).wait()
            pltpu.make_async_copy(ae0s[b], sa0_s.at[dsts[b]], sss[b]).wait()

        def _compute(b):
            for q in range(K // 16):
                sl = pl.ds(q * 16, 16)
                al = (plsc.load_gather(asrc_v, [srcc[b][sl]])
                      + plsc.load_gather(adst_v, [dstc[b][sl]]) + ae0c[b][sl])
                al = jnp.where(al > 0, al, 0.2 * al)
                e16 = jnp.exp(al)
                exc[b][sl] = e16
                dsts[b][sl] = dstc[b][sl]
                ae0s[b][sl] = ae0c[b][sl]
                for t in range(16):
                    e = q * 16 + t
                    w = e16[t]
                    for h in range(D // 16):
                        sl2 = pl.ds(h * 16, 16)
                        rows[b, e, sl2] = rows[b, e, sl2] * w

        def _scatter(b):
            pltpu.async_copy(rows.at[b], num_s.at[dsts[b]], sss[b], add=True)
            pltpu.async_copy(exc[b], den_s.at[dsts[b]], sss[b], add=True)
            pltpu.async_copy(ones_v, cnt_s.at[dsts[b]], sss[b], add=True)
            pltpu.async_copy(ae0s[b], sa0_s.at[dsts[b]], sss[b], add=True)

        _stage(0, 0)
        _stage(1, 1)
        _wait_stage(0)
        _gather(0)

        def _step(c, b, in_loop):
            nb = 1 - b
            if in_loop:
                _wait_stage(nb)

                @pl.when(c >= 1)
                def _():
                    _wait_scatter(nb)
                _gather(nb)
            _wait_gather(b)
            _compute(b)
            if in_loop:
                @pl.when(c <= NCHUNK - 3)
                def _():
                    _stage(c + 2, b)
            _scatter(b)

        def _pair(c2, _):
            for b in range(2):
                c = c2 * 2 + b
                _step(c, b, True)
            return 0
        lax.fori_loop(0, NCHUNK // 2, _pair, 0)
        _step(NCHUNK - 1, (NCHUNK - 1) % 2, False)
        _wait_scatter(0)
        _wait_scatter(1)

        plsc.subcore_barrier()

        pltpu.sync_copy(num_s.at[pl.ds(r0, RPT)],
                        num_h.at[cid, pl.ds(r0, RPT)])

        @pl.when(sid == NS - 1)
        def _():
            pltpu.sync_copy(num_s.at[pl.ds(NS * RPT, REM)],
                            num_h.at[cid, pl.ds(NS * RPT, REM)])

        @pl.when(sid == 0)
        def _():
            pltpu.sync_copy(den_s, den_h.at[cid])
            pltpu.sync_copy(cnt_s, cnt_h.at[cid])
            pltpu.sync_copy(sa0_s, sa0_h.at[cid])

    return body


def _merge_body(num_ref, den_ref, cnt_ref, sa0_ref, xw_ref, as_ref, ad_ref,
                b_ref, g_ref, be_ref, out_ref):
    den = den_ref[...]
    cnt = cnt_ref[...]
    sa0 = sa0_ref[...]
    ael = sa0 / jnp.maximum(cnt, 1.0)
    al = as_ref[...] + ad_ref[...] + ael
    al = jnp.where(al > 0, al, 0.2 * al)
    exl = jnp.exp(al)
    num = num_ref[0] + num_ref[1] + exl * xw_ref[...]
    out = num / (den + exl + 1e-16) + b_ref[...]
    mu = jnp.mean(out, axis=1, keepdims=True)
    var = jnp.mean((out - mu) ** 2, axis=1, keepdims=True)
    out_ref[...] = (out - mu) / jnp.sqrt(var + 1e-5) * g_ref[...] + be_ref[...]


def _merge(num, den, cnt, sa0, xw, a_src, a_dst, bias, gamma, beta, bn):
    _, N, D = num.shape
    den3 = den.sum(axis=0).reshape(N, 1)
    cnt3 = cnt.sum(axis=0).reshape(N, 1)
    sa03 = sa0.sum(axis=0).reshape(N, 1)
    return pl.pallas_call(
        _merge_body,
        grid=(N // bn,),
        in_specs=[
            pl.BlockSpec((NC, bn, D), lambda g: (0, g, 0)),
            pl.BlockSpec((bn, 1), lambda g: (g, 0)),
            pl.BlockSpec((bn, 1), lambda g: (g, 0)),
            pl.BlockSpec((bn, 1), lambda g: (g, 0)),
            pl.BlockSpec((bn, D), lambda g: (g, 0)),
            pl.BlockSpec((bn, 1), lambda g: (g, 0)),
            pl.BlockSpec((bn, 1), lambda g: (g, 0)),
            pl.BlockSpec((1, D), lambda g: (0, 0)),
            pl.BlockSpec((1, D), lambda g: (0, 0)),
            pl.BlockSpec((1, D), lambda g: (0, 0)),
        ],
        out_specs=pl.BlockSpec((bn, D), lambda g: (g, 0)),
        out_shape=jax.ShapeDtypeStruct((N, D), jnp.float32),
    )(num, den3, cnt3, sa03, xw, a_src, a_dst,
      bias.reshape(1, D), gamma.reshape(1, D), beta.reshape(1, D))


def kernel(x, edge_index, edge_attr, W, att_src, att_dst, W_edge, att_edge,
           bias, gamma, beta):
    N, D = x.shape
    E = edge_index.shape[1]

    src = edge_index[0].astype(jnp.int32)
    dst = edge_index[1].astype(jnp.int32)

    xw, a_src, a_dst = _node_scores(x, W, att_src, att_dst, bn=2000)
    ae0 = _edge_scores(edge_attr.T, W_edge, att_edge, cb=2560)

    sc = _sc_edge_kernel(N, D, E, K=80)
    num, den, cnt, sa0 = sc(src, dst, a_src.reshape(N), a_dst.reshape(N),
                            ae0.reshape(E), xw)

    return _merge(num, den, cnt, sa0, xw, a_src, a_dst,
                  bias, gamma, beta, bn=2000)


# speedup vs baseline: 1.1865x; 1.0343x over previous
import functools

import jax
import jax.numpy as jnp
from jax import lax
from jax.experimental import pallas as pl
from jax.experimental.pallas import tpu as pltpu
from jax.experimental.pallas import tpu_sc as plsc

NC = 2
NS = 16
NW = NC * NS


def _node_body(x_ref, w_ref, asrc_ref, adst_ref, xw_ref, as_ref, ad_ref):
    xw = jnp.dot(x_ref[...], w_ref[...], preferred_element_type=jnp.float32)
    xw_ref[...] = xw
    as_ref[...] = jnp.dot(xw, asrc_ref[...], preferred_element_type=jnp.float32)
    ad_ref[...] = jnp.dot(xw, adst_ref[...], preferred_element_type=jnp.float32)


def _node_scores(x, W, att_src, att_dst, bn):
    N, D = x.shape
    grid = (N // bn,)
    return pl.pallas_call(
        _node_body,
        grid=grid,
        in_specs=[
            pl.BlockSpec((bn, D), lambda g: (g, 0)),
            pl.BlockSpec((D, D), lambda g: (0, 0)),
            pl.BlockSpec((D, 1), lambda g: (0, 0)),
            pl.BlockSpec((D, 1), lambda g: (0, 0)),
        ],
        out_specs=[
            pl.BlockSpec((bn, D), lambda g: (g, 0)),
            pl.BlockSpec((bn, 1), lambda g: (g, 0)),
            pl.BlockSpec((bn, 1), lambda g: (g, 0)),
        ],
        out_shape=[
            jax.ShapeDtypeStruct((N, D), jnp.float32),
            jax.ShapeDtypeStruct((N, 1), jnp.float32),
            jax.ShapeDtypeStruct((N, 1), jnp.float32),
        ],
    )(x, W, att_src.reshape(D, 1), att_dst.reshape(D, 1))


def _edge_body(E, CB, eat_ref, we_ref, ae_ref, out_ref):
    v2 = jnp.dot(we_ref[...], ae_ref[...], preferred_element_type=jnp.float32)
    rpc = CB // 128

    def step(c, _):
        chunk = eat_ref[:, pl.ds(c * CB, CB)]
        s = jnp.sum(chunk * v2, axis=0, keepdims=True)
        for m in range(rpc):
            out_ref[pl.ds(c * rpc + m, 1), :] = s[:, m * 128:(m + 1) * 128]
        return 0
    lax.fori_loop(0, E // CB, step, 0)


def _edge_scores(edge_attr_t, W_edge, att_edge, cb):
    DE, E = edge_attr_t.shape
    D = W_edge.shape[1]
    return pl.pallas_call(
        functools.partial(_edge_body, E, cb),
        in_specs=[
            pl.BlockSpec((DE, E), lambda: (0, 0)),
            pl.BlockSpec((DE, D), lambda: (0, 0)),
            pl.BlockSpec((D, 1), lambda: (0, 0)),
        ],
        out_specs=pl.BlockSpec((E // 128, 128), lambda: (0, 0)),
        out_shape=jax.ShapeDtypeStruct((E // 128, 128), jnp.float32),
    )(edge_attr_t, W_edge, att_edge.reshape(D, 1))


def _sc_edge_kernel(N, D, E, K):
    EPW = E // NW
    P1 = EPW // 16
    NCHUNK = EPW // K
    RPT = (N // NS) // 8 * 8
    REM = N - NS * RPT
    ZB = K

    mesh = plsc.VectorSubcoreMesh(core_axis_name="c", subcore_axis_name="s")

    @functools.partial(
        pl.kernel,
        out_type=(
            jax.ShapeDtypeStruct((NC, N, D), jnp.float32),
            jax.ShapeDtypeStruct((NC, N), jnp.float32),
            jax.ShapeDtypeStruct((NC, N), jnp.float32),
            jax.ShapeDtypeStruct((NC, N), jnp.float32),
        ),
        mesh=mesh,
        scratch_types=[
            pltpu.VMEM((K,), jnp.int32),
            pltpu.VMEM((K,), jnp.int32),
            pltpu.VMEM((K,), jnp.int32),
            pltpu.VMEM((K,), jnp.int32),
            pltpu.VMEM((K,), jnp.float32),
            pltpu.VMEM((K,), jnp.float32),
            pltpu.VMEM((K,), jnp.int32),
            pltpu.VMEM((K,), jnp.int32),
            pltpu.VMEM((K,), jnp.float32),
            pltpu.VMEM((K,), jnp.float32),
            pltpu.VMEM((N,), jnp.float32),
            pltpu.VMEM((N,), jnp.float32),
            pltpu.VMEM((K,), jnp.float32),
            pltpu.VMEM((K,), jnp.float32),
            pltpu.VMEM((K,), jnp.float32),
            pltpu.VMEM((RPT,), jnp.float32),
            pltpu.VMEM((2, K, D), jnp.float32),
            pltpu.VMEM_SHARED((N, D), jnp.float32),
            pltpu.VMEM_SHARED((N,), jnp.float32),
            pltpu.VMEM_SHARED((N,), jnp.float32),
            pltpu.VMEM_SHARED((N,), jnp.float32),
            pltpu.SemaphoreType.DMA,
            pltpu.SemaphoreType.DMA,
            pltpu.SemaphoreType.DMA,
            pltpu.SemaphoreType.DMA,
            pltpu.SemaphoreType.DMA,
            pltpu.SemaphoreType.DMA,
        ],
        compiler_params=pltpu.CompilerParams(needs_layout_passes=False),
    )
    def body(src_h, dst_h, asrc_h, adst_h, ae0_h, xw_h,
             num_h, den_h, cnt_h, sa0_h,
             srcc0, srcc1, dstc0, dstc1, ae0c0, ae0c1, dsts0, dsts1,
             ae0s0, ae0s1, asrc_v, adst_v, exc0, exc1, ones_v, zer_v,
             rows, num_s, den_s, cnt_s, sa0_s,
             sg0, sg1, gg0, gg1, ss0, ss1):
        cid = lax.axis_index("c")
        sid = lax.axis_index("s")
        wid = cid * NS + sid
        base = wid * EPW
        srcc = (srcc0, srcc1)
        dstc = (dstc0, dstc1)
        ae0c = (ae0c0, ae0c1)
        dsts = (dsts0, dsts1)
        ae0s = (ae0s0, ae0s1)
        exc = (exc0, exc1)
        sgs = (sg0, sg1)
        ggs = (gg0, gg1)
        sss = (ss0, ss1)

        zero16 = jnp.zeros((16,), jnp.float32)
        one16 = jnp.ones((16,), jnp.float32)

        def _zinit(j, _):
            zer_v[pl.ds(j * 16, 16)] = zero16
            return 0
        lax.fori_loop(0, RPT // 16, _zinit, 0)
        for q in range(K // 16):
            ones_v[pl.ds(q * 16, 16)] = one16

        def _zrow(e, _):
            for h in range(D // 16):
                rows[0, e, pl.ds(h * 16, 16)] = zero16
            return 0
        lax.fori_loop(0, K, _zrow, 0)

        pltpu.sync_copy(asrc_h, asrc_v)
        pltpu.sync_copy(adst_h, adst_v)

        r0 = sid * RPT
        for q in range((RPT + ZB - 1) // ZB):
            sz = min(ZB, RPT - q * ZB)
            pltpu.sync_copy(rows.at[0, pl.ds(0, sz)],
                            num_s.at[pl.ds(r0 + q * ZB, sz)])
        pltpu.sync_copy(zer_v, den_s.at[pl.ds(r0, RPT)])
        pltpu.sync_copy(zer_v, cnt_s.at[pl.ds(r0, RPT)])
        pltpu.sync_copy(zer_v, sa0_s.at[pl.ds(r0, RPT)])

        @pl.when(sid == NS - 1)
        def _():
            pltpu.sync_copy(rows.at[0, pl.ds(0, REM)],
                            num_s.at[pl.ds(NS * RPT, REM)])
            pltpu.sync_copy(zer_v.at[pl.ds(0, REM)],
                            den_s.at[pl.ds(NS * RPT, REM)])
            pltpu.sync_copy(zer_v.at[pl.ds(0, REM)],
                            cnt_s.at[pl.ds(NS * RPT, REM)])
            pltpu.sync_copy(zer_v.at[pl.ds(0, REM)],
                            sa0_s.at[pl.ds(NS * RPT, REM)])
        plsc.subcore_barrier()

        def _stage(c, b):
            gb = base + c * K
            pltpu.async_copy(src_h.at[pl.ds(gb, K)], srcc[b], sgs[b])
            pltpu.async_copy(dst_h.at[pl.ds(gb, K)], dstc[b], sgs[b])
            pltpu.async_copy(ae0_h.at[pl.ds(gb, K)], ae0c[b], sgs[b])

        def _wait_stage(b):
            pltpu.make_async_copy(src_h.at[pl.ds(0, K)], srcc[b], sgs[b]).wait()
            pltpu.make_async_copy(dst_h.at[pl.ds(0, K)], dstc[b], sgs[b]).wait()
            pltpu.make_async_copy(ae0_h.at[pl.ds(0, K)], ae0c[b], sgs[b]).wait()

        def _gather(b):
            pltpu.async_copy(xw_h.at[srcc[b]], rows.at[b], ggs[b])

        def _wait_gather(b):
            pltpu.make_async_copy(xw_h.at[srcc[b]], rows.at[b], ggs[b]).wait()

        def _wait_scatter(b):
            pltpu.make_async_copy(rows.at[b], num_s.at[dsts[b]], sss[b]).wait()
            pltpu.make_async_copy(exc[b], den_s.at[dsts[b]], sss[b]).wait()
            pltpu.make_async_copy(ones_v, cnt_s.at[dsts[b]], sss[b]).wait()
            pltpu.make_async_copy(ae0s[b], sa0_s.at[dsts[b]], sss[b]).wait()

        def _compute(b):
            for q in range(K // 16):
                sl = pl.ds(q * 16, 16)
                al = (plsc.load_gather(asrc_v, [srcc[b][sl]])
                      + plsc.load_gather(adst_v, [dstc[b][sl]]) + ae0c[b][sl])
                al = jnp.where(al > 0, al, 0.2 * al)
                e16 = jnp.exp(al)
                exc[b][sl] = e16
                dsts[b][sl] = dstc[b][sl]
                ae0s[b][sl] = ae0c[b][sl]
                for t in range(16):
                    e = q * 16 + t
                    w = e16[t]
                    for h in range(D // 16):
                        sl2 = pl.ds(h * 16, 16)
                        rows[b, e, sl2] = rows[b, e, sl2] * w

        def _scatter(b):
            pltpu.async_copy(rows.at[b], num_s.at[dsts[b]], sss[b], add=True)
            pltpu.async_copy(exc[b], den_s.at[dsts[b]], sss[b], add=True)
            pltpu.async_copy(ones_v, cnt_s.at[dsts[b]], sss[b], add=True)
            pltpu.async_copy(ae0s[b], sa0_s.at[dsts[b]], sss[b], add=True)

        _stage(0, 0)
        _stage(1, 1)
        _wait_stage(0)
        _gather(0)

        def _step(c, b, in_loop):
            nb = 1 - b
            if in_loop:
                _wait_stage(nb)

                @pl.when(c >= 1)
                def _():
                    _wait_scatter(nb)
                _gather(nb)
            _wait_gather(b)
            _compute(b)
            if in_loop:
                @pl.when(c <= NCHUNK - 3)
                def _():
                    _stage(c + 2, b)
            _scatter(b)

        def _pair(c2, _):
            for b in range(2):
                c = c2 * 2 + b
                _step(c, b, True)
            return 0
        lax.fori_loop(0, NCHUNK // 2, _pair, 0)
        _step(NCHUNK - 1, (NCHUNK - 1) % 2, False)
        _wait_scatter(0)
        _wait_scatter(1)

        plsc.subcore_barrier()

        pltpu.sync_copy(num_s.at[pl.ds(r0, RPT)],
                        num_h.at[cid, pl.ds(r0, RPT)])

        @pl.when(sid == NS - 1)
        def _():
            pltpu.sync_copy(num_s.at[pl.ds(NS * RPT, REM)],
                            num_h.at[cid, pl.ds(NS * RPT, REM)])

        @pl.when(sid == 0)
        def _():
            pltpu.sync_copy(den_s, den_h.at[cid])
            pltpu.sync_copy(cnt_s, cnt_h.at[cid])
            pltpu.sync_copy(sa0_s, sa0_h.at[cid])

    return body


def _merge_body(num_ref, den_ref, cnt_ref, sa0_ref, xw_ref, as_ref, ad_ref,
                b_ref, g_ref, be_ref, out_ref):
    den = den_ref[...]
    cnt = cnt_ref[...]
    sa0 = sa0_ref[...]
    ael = sa0 / jnp.maximum(cnt, 1.0)
    al = as_ref[...] + ad_ref[...] + ael
    al = jnp.where(al > 0, al, 0.2 * al)
    exl = jnp.exp(al)
    num = num_ref[0] + num_ref[1] + exl * xw_ref[...]
    out = num / (den + exl + 1e-16) + b_ref[...]
    mu = jnp.mean(out, axis=1, keepdims=True)
    var = jnp.mean((out - mu) ** 2, axis=1, keepdims=True)
    out_ref[...] = (out - mu) / jnp.sqrt(var + 1e-5) * g_ref[...] + be_ref[...]


def _merge(num, den, cnt, sa0, xw, a_src, a_dst, bias, gamma, beta, bn):
    _, N, D = num.shape
    den3 = den.sum(axis=0).reshape(N, 1)
    cnt3 = cnt.sum(axis=0).reshape(N, 1)
    sa03 = sa0.sum(axis=0).reshape(N, 1)
    return pl.pallas_call(
        _merge_body,
        grid=(N // bn,),
        in_specs=[
            pl.BlockSpec((NC, bn, D), lambda g: (0, g, 0)),
            pl.BlockSpec((bn, 1), lambda g: (g, 0)),
            pl.BlockSpec((bn, 1), lambda g: (g, 0)),
            pl.BlockSpec((bn, 1), lambda g: (g, 0)),
            pl.BlockSpec((bn, D), lambda g: (g, 0)),
            pl.BlockSpec((bn, 1), lambda g: (g, 0)),
            pl.BlockSpec((bn, 1), lambda g: (g, 0)),
            pl.BlockSpec((1, D), lambda g: (0, 0)),
            pl.BlockSpec((1, D), lambda g: (0, 0)),
            pl.BlockSpec((1, D), lambda g: (0, 0)),
        ],
        out_specs=pl.BlockSpec((bn, D), lambda g: (g, 0)),
        out_shape=jax.ShapeDtypeStruct((N, D), jnp.float32),
    )(num, den3, cnt3, sa03, xw, a_src, a_dst,
      bias.reshape(1, D), gamma.reshape(1, D), beta.reshape(1, D))


def kernel(x, edge_index, edge_attr, W, att_src, att_dst, W_edge, att_edge,
           bias, gamma, beta):
    N, D = x.shape
    E = edge_index.shape[1]

    src = edge_index[0].astype(jnp.int32)
    dst = edge_index[1].astype(jnp.int32)

    xw, a_src, a_dst = _node_scores(x, W, att_src, att_dst, bn=2000)
    ae0 = _edge_scores(edge_attr.T, W_edge, att_edge, cb=2560)

    sc = _sc_edge_kernel(N, D, E, K=80)
    num, den, cnt, sa0 = sc(src, dst, a_src.reshape(N), a_dst.reshape(N),
                            ae0.reshape(E), xw)

    return _merge(num, den, cnt, sa0, xw, a_src, a_dst,
                  bias, gamma, beta, bn=2000)
